# baseline (speedup 1.0000x reference)
import functools

import jax
import jax.numpy as jnp
from jax import lax
from jax.experimental import pallas as pl
from jax.experimental.pallas import tpu as pltpu
from jax.experimental.pallas import tpu_sc as plsc

_D = 64
_L = 128
_K = 8
_NC = 2
_NS = 16
_NW = _NC * _NS

_mesh = plsc.VectorSubcoreMesh(core_axis_name="c", subcore_axis_name="s")


def _make_gather(n_rows: int):
    rows_per_w = n_rows // _NW
    n_steps = rows_per_w // _K

    @functools.partial(
        pl.kernel,
        out_type=jax.ShapeDtypeStruct((n_rows, _L, _D), jnp.float32),
        mesh=_mesh,
        scratch_types=[
            pltpu.VMEM((_K, _L), jnp.int32),
            pltpu.VMEM((_K, _L, _D), jnp.float32),
            pltpu.SemaphoreType.DMA,
        ],
        compiler_params=pltpu.CompilerParams(use_tc_tiling_on_sc=False),
    )
    def _gather(idx_hbm, table_hbm, out_hbm, idx_v, rows_v, sem):
        wid = lax.axis_index("s") * _NC + lax.axis_index("c")
        base = wid * rows_per_w

        def step(i, carry):
            r = base + i * _K
            pltpu.sync_copy(idx_hbm.at[pl.ds(r, _K)], idx_v)
            waits = []
            for j in range(_K):
                waits.append(
                    pltpu.async_copy(table_hbm.at[idx_v.at[j]], rows_v.at[j], sem)
                )
            for w in waits:
                w.wait()
            pltpu.sync_copy(rows_v, out_hbm.at[pl.ds(r, _K)])
            return carry

        lax.fori_loop(0, n_steps, step, 0)

    return _gather


def kernel(x, table):
    b, h = x.shape
    flat = x.reshape(-1).astype(jnp.int32)
    n_rows = flat.shape[0] // _L
    idx2d = flat.reshape(n_rows, _L)
    out = _make_gather(n_rows)(idx2d, table)
    return out.reshape(b, h, _D)


# speedup vs baseline: 1.0126x; 1.0126x over previous
import functools

import jax
import jax.numpy as jnp
from jax import lax
from jax.experimental import pallas as pl
from jax.experimental.pallas import tpu as pltpu
from jax.experimental.pallas import tpu_sc as plsc

_D = 64
_L = 128
_K = 5
_NC = 2
_NS = 16
_NW = _NC * _NS

_mesh = plsc.VectorSubcoreMesh(core_axis_name="c", subcore_axis_name="s")


def _make_gather(n_rows: int):
    rows_per_w = n_rows // _NW
    n_steps = rows_per_w // _K

    @functools.partial(
        pl.kernel,
        out_type=jax.ShapeDtypeStruct((n_rows, _L, _D), jnp.float32),
        mesh=_mesh,
        scratch_types=[
            pltpu.VMEM((rows_per_w, _L), jnp.int32),
            pltpu.VMEM((_K, _L, _D), jnp.float32),
            pltpu.VMEM((_K, _L, _D), jnp.float32),
            pltpu.SemaphoreType.DMA,
            pltpu.SemaphoreType.DMA,
            pltpu.SemaphoreType.DMA,
            pltpu.SemaphoreType.DMA,
        ],
        compiler_params=pltpu.CompilerParams(use_tc_tiling_on_sc=False),
    )
    def _gather(idx_hbm, table_hbm, out_hbm, idx_v, buf0, buf1,
                sg0, sg1, sw0, sw1):
        wid = lax.axis_index("s") * _NC + lax.axis_index("c")
        base = wid * rows_per_w
        pltpu.sync_copy(idx_hbm.at[pl.ds(base, rows_per_w)], idx_v)

        bufs = (buf0, buf1)
        sgs = (sg0, sg1)
        sws = (sw0, sw1)

        def g_start(i, b):
            for j in range(_K):
                pltpu.async_copy(
                    table_hbm.at[idx_v.at[i * _K + j]],
                    bufs[b].at[j], sgs[b])

        def g_wait(b):
            pltpu.make_async_copy(
                out_hbm.at[pl.ds(base, _K)], bufs[b], sgs[b]).wait()

        def w_desc(i, b):
            return pltpu.make_async_copy(
                bufs[b], out_hbm.at[pl.ds(base + i * _K, _K)], sws[b])

        g_start(0, 0)
        g_start(1, 1)

        def body(j, carry):
            for b in range(2):
                i = j * 2 + b
                g_wait(b)
                w_desc(i, b).start()
                w_desc(i, b).wait()

                @pl.when(i + 2 < n_steps)
                def _():
                    g_start(i + 2, b)

            return carry

        lax.fori_loop(0, n_steps // 2, body, 0)

    return _gather


def kernel(x, table):
    b, h = x.shape
    flat = x.reshape(-1).astype(jnp.int32)
    n_rows = flat.shape[0] // _L
    idx2d = flat.reshape(n_rows, _L)
    out = _make_gather(n_rows)(idx2d, table)
    return out.reshape(b, h, _D)
